# baseline (speedup 1.0000x reference)
import functools
import math

import jax
import jax.numpy as jnp
from jax import lax
from jax.experimental import pallas as pl
from jax.experimental.pallas import tpu as pltpu
from jax.experimental.pallas import tpu_sc as plsc

_NC, _NS = 2, 16
_NW = _NC * _NS
_IDXW = 128
_NSUB = 8
_CHUNK = _IDXW * _NSUB


@functools.partial(jax.jit, static_argnums=(2, 3))
def _emb(xf, lut, B, D):
    b_per_w = B // _NW
    n_chunks = b_per_w // _CHUNK
    scale = math.sqrt(float(D))
    mesh = plsc.VectorSubcoreMesh(core_axis_name="c", subcore_axis_name="s")

    @functools.partial(
        pl.kernel,
        mesh=mesh,
        out_type=jax.ShapeDtypeStruct((B, D), jnp.float32),
        scratch_types=[
            pltpu.VMEM((_NSUB, _IDXW), jnp.int32),
            pltpu.VMEM((_CHUNK, D), jnp.float32),
            pltpu.SemaphoreType.DMA,
        ],
        compiler_params=pltpu.CompilerParams(use_tc_tiling_on_sc=False),
    )
    def emb_kernel(x_hbm, lut_hbm, out_hbm, idx_v, rows_v, sem):
        wid = lax.axis_index("s") * _NC + lax.axis_index("c")
        base = wid * b_per_w

        def chunk_body(ci, carry):
            off = pl.multiple_of(base + ci * _CHUNK, _CHUNK)
            row0 = pl.multiple_of(off // _IDXW, _NSUB)
            pltpu.sync_copy(x_hbm.at[pl.ds(row0, _NSUB)], idx_v)
            copies = [
                pltpu.async_copy(
                    lut_hbm.at[idx_v.at[j]],
                    rows_v.at[pl.ds(j * _IDXW, _IDXW)],
                    sem,
                )
                for j in range(_NSUB)
            ]
            for c in copies:
                c.wait()

            def mul_body(r, mc):
                for q in range(D // 16):
                    v = rows_v[r, pl.ds(q * 16, 16)]
                    rows_v[r, pl.ds(q * 16, 16)] = v * scale
                return mc

            lax.fori_loop(0, _CHUNK, mul_body, 0)
            pltpu.sync_copy(rows_v, out_hbm.at[pl.ds(off, _CHUNK)])
            return carry

        lax.fori_loop(0, n_chunks, chunk_body, 0)

    return emb_kernel(xf, lut)


def kernel(x, lut):
    bt, s = x.shape
    b = bt * s
    d = lut.shape[1]
    xf = x.reshape(b // _IDXW, _IDXW).astype(jnp.int32)
    out = _emb(xf, lut, b, d)
    return out.reshape(bt, s, d)
